# baseline (speedup 1.0000x reference)
import functools

import numpy as np
import jax
import jax.numpy as jnp
from jax.experimental import pallas as pl
from jax.experimental.pallas import tpu as pltpu

_B, _N = 2, 8
_VOX = 16 * 16 * 16
_NPAIR_U = 36
_NV = _B * _N + _B * _NPAIR_U
_VB = 8
_F32 = jnp.float32
_BF16 = jnp.bfloat16


def _split(a):
    ah = a.astype(_BF16)
    al = (a - ah.astype(_F32)).astype(_BF16)
    return ah, al


def _d(x, y, dn):
    return jax.lax.dot_general(x, y, dn, preferred_element_type=_F32)


def _dmm(ah, al, bh, bl, dn):
    return _d(ah, bh, dn) + (_d(ah, bl, dn) + _d(al, bh, dn))


def _dot3(a, b, dn):
    ah, al = _split(a)
    bh, bl = _split(b)
    return _dmm(ah, al, bh, bl, dn)


def _dsel(s, b, dn):
    bh, bl = _split(b)
    return _d(s, bh, dn) + _d(s, bl, dn)


def _dsel_r(a, s, dn):
    ah, al = _split(a)
    return _d(ah, s, dn) + _d(al, s, dn)


_PAIRS = [(j, k) for j in range(8) for k in range(j, 8)]


def _sel_constants():
    sj = np.zeros((_B * _NPAIR_U, _B * _N), np.float32)
    sk = np.zeros((_B * _NPAIR_U, _B * _N), np.float32)
    em = np.zeros((_NPAIR_U, _N * _N), np.float32)
    for u, (j, k) in enumerate(_PAIRS):
        em[u, j * 8 + k] = 1.0
        em[u, k * 8 + j] = 1.0
        for i in range(_B):
            sj[i * _NPAIR_U + u, i * 8 + j] = 1.0
            sk[i * _NPAIR_U + u, i * 8 + k] = 1.0
    gs = np.zeros((128, 32), np.float32)
    for c in range(128):
        gs[c, c // 4] = 1.0
    return sj, sk, em, gs


def _clip1(x):
    return x - jax.nn.relu(x - 1.0)


def _build_pairs(base, sj, sk):
    dn = (((1,), (0,)), ((), ()))
    bh, bl = _split(base)
    pj = _d(sj, bh, dn) + _d(sj, bl, dn)
    pk = _d(sk, bh, dn) + _d(sk, bl, dn)
    return _clip1(pj + pk)


def _prep_body(m_ref, sj_ref, sk_ref, o_ref):
    m2 = m_ref[...]
    o_ref[0:16, :] = m2
    o_ref[16:_NV, :] = _build_pairs(m2, sj_ref[...], sk_ref[...])


def _gn_stats_from_sums(s1, s2, npos, gs):
    cnt = 4.0 * npos
    mu = _dsel_r(s1, gs, (((1,), (0,)), ((), ()))) / cnt
    ex2 = _dsel_r(s2, gs, (((1,), (0,)), ((), ()))) / cnt
    var = ex2 - mu * mu
    inv = jax.lax.rsqrt(var + 1e-5)
    mu_b = _dsel_r(mu, gs, (((1,), (1,)), ((), ())))
    inv_b = _dsel_r(inv, gs, (((1,), (1,)), ((), ())))
    return mu_b, inv_b


def _gn_apply_silu(y, mu_b, inv_b, g, bt):
    yn = (y - mu_b[:, None, :]) * inv_b[:, None, :]
    yn = yn * g[None, :, :] + bt[None, :, :]
    return yn * jax.nn.sigmoid(yn)


_CLASSES = [(cx, cy, cz) for cx in range(2) for cy in range(2)
            for cz in range(2)]
_CLS_N = [(4 - c[0]) * (4 - c[1]) * (4 - c[2]) for c in _CLASSES]
_CLS_OFF = [sum(_CLS_N[:i]) for i in range(len(_CLASSES))]

_POS_LIST = [(2 * ux + cx, 2 * uy + cy, 2 * uz + cz)
             for (cx, cy, cz) in _CLASSES
             for ux in range(4 - cx) for uy in range(4 - cy)
             for uz in range(4 - cz)]
_IM2COL_IDX = np.array(
    [[(2 * i + dx) * 256 + (2 * j + dy) * 16 + (2 * k + dz)
      for dx in range(3) for dy in range(3) for dz in range(3)]
     for (i, j, k) in _POS_LIST], dtype=np.int32)

_DN2 = (((2,), (0,)), ((), ()))


def _mapper_body(x1h_ref, x1l_ref, w1h_ref, w1l_ref, w2h_ref, w2l_ref,
                 w3h_ref, w3l_ref, prm_ref, gs_ref, o_ref, sc1, sc2):
    vb = x1h_ref.shape[0]
    prm = prm_ref[...]
    gs = gs_ref[...]

    y1 = _dmm(x1h_ref[...], x1l_ref[...], w1h_ref[...], w1l_ref[...], _DN2)
    sc1[...] = y1 + prm[0:1, :][None]

    def stats(ref, npos):
        s1 = jnp.zeros((vb, 128), _F32)
        s2 = jnp.zeros((vb, 128), _F32)
        for ci in range(8):
            off, n = _CLS_OFF[ci], _CLS_N[ci]
            y = ref[:, off:off + n, :]
            s1 = s1 + y.sum(axis=1)
            s2 = s2 + (y * y).sum(axis=1)
        return _gn_stats_from_sums(s1, s2, npos, gs)

    mu_b, inv_b = stats(sc1, 343.0)
    w2h, w2l = w2h_ref[...], w2l_ref[...]
    for ci in range(8):
        off, n = _CLS_OFF[ci], _CLS_N[ci]
        y = _gn_apply_silu(sc1[:, off:off + n, :], mu_b, inv_b,
                           prm[1:2, :], prm[2:3, :])
        yh, yl = _split(y)
        y2c = _dmm(yh, yl, w2h, w2l, _DN2)
        sc2[:, off:off + n, :] = y2c + prm[3:4, :][None]

    mu_b, inv_b = stats(sc2, 343.0)
    acc = jnp.zeros((vb, 27, 128), _F32)
    for ci, (cx, cy, cz) in enumerate(_CLASSES):
        off, n = _CLS_OFF[ci], _CLS_N[ci]
        lx, ly, lz = 4 - cx, 4 - cy, 4 - cz
        y = _gn_apply_silu(sc2[:, off:off + n, :], mu_b, inv_b,
                           prm[4:5, :], prm[5:6, :])
        y5 = y.reshape(vb, lx, ly, lz, 128)
        yh, yl = _split(y5)
        for dx in range(cx, 3, 2):
            for dy in range(cy, 3, 2):
                for dz in range(cz, 3, 2):
                    t = (dx * 3 + dy) * 3 + dz
                    ix = (slice(None), slice(dx // 2, dx // 2 + 3),
                          slice(dy // 2, dy // 2 + 3),
                          slice(dz // 2, dz // 2 + 3), slice(None))
                    acc = acc + _dmm(
                        yh[ix].reshape(vb, 27, 128),
                        yl[ix].reshape(vb, 27, 128),
                        w3h_ref[t], w3l_ref[t], _DN2)
    y3 = acc + prm[6:7, :][None]
    s1 = y3.sum(axis=1)
    s2 = (y3 * y3).sum(axis=1)
    mu_b, inv_b = _gn_stats_from_sums(s1, s2, 27.0, gs)
    y3 = _gn_apply_silu(y3, mu_b, inv_b, prm[7:8, :], prm[8:9, :])
    o_ref[...] = jnp.mean(y3, axis=1)


def _codebook_p(q, cb):
    d = q[:, None, :] - cb[None, :, :]
    sq = jnp.sum(d * d, axis=-1)
    return jnp.exp(-jnp.sqrt(jnp.maximum(sq, 1e-12)))


def _head_body(with_vox, f_ref, base_ref, qswh_ref, qswl_ref, qsb_ref,
               qpwh_ref, qpwl_ref, qpb_ref,
               pa_ref, pb_ref, sj_ref, sk_ref, em_ref, *out_refs):
    f = f_ref[...]
    base = base_ref[...]
    dn_t = (((1,), (1,)), ((), ()))
    f1h, f1l = _split(f[0:16, :])
    f2h, f2l = _split(f[16:_NV, :])
    q1 = _dmm(f1h, f1l, qswh_ref[...], qswl_ref[...], dn_t) + qsb_ref[...]
    q2 = _dmm(f2h, f2l, qpwh_ref[...], qpwl_ref[...], dn_t) + qpb_ref[...]
    pa = pa_ref[...]
    p1 = _codebook_p(q1, pa)
    out_refs[0][...] = _dot3(p1, pa, (((1,), (0,)), ((), ())))
    d2 = q2[:, None, :] - pb_ref[...][None, :, :]
    sq2 = jnp.sum(d2 * d2, axis=-1)
    p2c = jnp.exp(-jnp.sqrt(jnp.maximum(sq2, 1e-12)))
    ci = jax.lax.broadcasted_iota(jnp.int32, (12, 6), 0)
    gi = jax.lax.broadcasted_iota(jnp.int32, (12, 6), 1)
    sum2c = ((ci // 2) == gi).astype(_BF16)
    s2c = _dsel_r(p2c, sum2c, (((1,), (0,)), ((), ())))
    cols = [s2c[:, i:i + 1] for i in range(6)]
    num_c = cols[0]
    den_c = cols[0]
    for r in cols[1:]:
        num_c = jnp.maximum(num_c, r)
        den_c = den_c + r
    em = em_ref[...]
    nm = _dsel_r(num_c.reshape(_B, _NPAIR_U), em,
                 (((1,), (0,)), ((), ())))
    dn = _dsel_r(den_c.reshape(_B, _NPAIR_U), em,
                 (((1,), (0,)), ((), ())))
    pm = (nm / dn).reshape(_B, 8, 8)
    z8 = jnp.zeros((8, 8), _F32)
    p2m16 = jnp.concatenate(
        [jnp.concatenate([pm[0], z8], axis=1),
         jnp.concatenate([z8, pm[1]], axis=1)], axis=0)
    g2 = _dot3(p2m16, base, (((1,), (0,)), ((), ())))
    out_refs[1][...] = g2
    if with_vox:
        out_refs[2][0:16, :] = g2
        out_refs[2][16:_NV, :] = _build_pairs(g2, sj_ref[...], sk_ref[...])


def _run_mapper(vox, w1s, w2s, w3s, prm, gs):
    x1 = vox[:, jnp.asarray(_IM2COL_IDX)]
    x1h, x1l = _split(x1)
    nblk = _NV // _VB
    bs_w = [
        pl.BlockSpec((27, 128), lambda i: (0, 0)),
        pl.BlockSpec((27, 128), lambda i: (0, 0)),
        pl.BlockSpec((128, 128), lambda i: (0, 0)),
        pl.BlockSpec((128, 128), lambda i: (0, 0)),
        pl.BlockSpec((27, 128, 128), lambda i: (0, 0, 0)),
        pl.BlockSpec((27, 128, 128), lambda i: (0, 0, 0)),
        pl.BlockSpec((9, 128), lambda i: (0, 0)),
        pl.BlockSpec((128, 32), lambda i: (0, 0)),
    ]
    return pl.pallas_call(
        _mapper_body,
        grid=(nblk,),
        in_specs=[
            pl.BlockSpec((_VB, 343, 27), lambda i: (i, 0, 0)),
            pl.BlockSpec((_VB, 343, 27), lambda i: (i, 0, 0)),
        ] + bs_w,
        out_specs=pl.BlockSpec((_VB, 128), lambda i: (i, 0)),
        out_shape=jax.ShapeDtypeStruct((_NV, 128), _F32),
        scratch_shapes=[pltpu.VMEM((_VB, 343, 128), _F32),
                        pltpu.VMEM((_VB, 343, 128), _F32)],
    )(x1h, x1l, *w1s, *w2s, *w3s, prm, gs)


def _run_head(with_vox, f, base, qsw_s, qsb, qpw_s, qpb, pa, pb, sj, sk, em):
    outs = [jax.ShapeDtypeStruct((16, 256), _F32),
            jax.ShapeDtypeStruct((16, _VOX), _F32)]
    if with_vox:
        outs.append(jax.ShapeDtypeStruct((_NV, _VOX), _F32))
    return pl.pallas_call(
        functools.partial(_head_body, with_vox),
        out_shape=outs,
    )(f, base, *qsw_s, qsb, *qpw_s, qpb, pa, pb, sj, sk, em)


def kernel(m, W1, b1, g1, bt1, W2, b2, g2, bt2, W3, b3, g3, bt3,
           Q1_W, Q1_b, Q2_W, Q2_b, QH1_W, QH1_b, QH2_W, QH2_b,
           P1, P2, PH1, PH2):
    m2d = m.reshape(_B * _N, _VOX)
    w1s = _split(W1.transpose(2, 3, 4, 1, 0).reshape(27, 128))
    w2s = _split(W2[:, :, 0, 0, 0].transpose(1, 0))
    w3s = _split(W3.transpose(2, 3, 4, 1, 0).reshape(27, 128, 128))
    prm = jnp.stack([b1, g1, bt1, b2, g2, bt2, b3, g3, bt3])
    sj_np, sk_np, em_np, gs_np = _sel_constants()
    sj = jnp.asarray(sj_np, _BF16)
    sk = jnp.asarray(sk_np, _BF16)
    em = jnp.asarray(em_np, _BF16)
    gs = jnp.asarray(gs_np, _BF16)
    q1s, q2s = _split(Q1_W), _split(Q2_W)
    qh1s, qh2s = _split(QH1_W), _split(QH2_W)
    q1b, q2b = Q1_b.reshape(1, 256), Q2_b.reshape(1, 256)
    qh1b, qh2b = QH1_b.reshape(1, 256), QH2_b.reshape(1, 256)

    vox_a = pl.pallas_call(
        _prep_body,
        out_shape=jax.ShapeDtypeStruct((_NV, _VOX), _F32),
    )(m2d, sj, sk)
    feats_a = _run_mapper(vox_a, w1s, w2s, w3s, prm, gs)
    pr16, g2d, vox_b = _run_head(True, feats_a, m2d, q1s, q1b, q2s, q2b,
                                 P1, P2, sj, sk, em)
    feats_b = _run_mapper(vox_b, w1s, w2s, w3s, prm, gs)
    phr16, gh2d = _run_head(False, feats_b, g2d, qh1s, qh1b, qh2s, qh2b,
                            PH1, PH2, sj, sk, em)

    pr = pr16.reshape(_B, _N, 256)
    gr = g2d.reshape(_B, _N, 1, 16, 16, 16)
    phr = phr16.reshape(_B, _N, 256)
    ghr = gh2d.reshape(_B, _N, 1, 16, 16, 16)
    return pr, gr, phr, ghr


# speedup vs baseline: 1.0997x; 1.0997x over previous
import functools

import numpy as np
import jax
import jax.numpy as jnp
from jax.experimental import pallas as pl
from jax.experimental.pallas import tpu as pltpu

_B, _N = 2, 8
_VOX = 16 * 16 * 16
_NPAIR_U = 36
_NV = _B * _N + _B * _NPAIR_U
_VB = 8
_F32 = jnp.float32
_BF16 = jnp.bfloat16


def _split(a):
    ah = a.astype(_BF16)
    al = (a - ah.astype(_F32)).astype(_BF16)
    return ah, al


def _d(x, y, dn):
    return jax.lax.dot_general(x, y, dn, preferred_element_type=_F32)


def _dmm(ah, al, bh, bl, dn):
    return _d(ah, bh, dn) + (_d(ah, bl, dn) + _d(al, bh, dn))


def _dot3(a, b, dn):
    ah, al = _split(a)
    bh, bl = _split(b)
    return _dmm(ah, al, bh, bl, dn)


def _dsel(s, b, dn):
    bh, bl = _split(b)
    return _d(s, bh, dn) + _d(s, bl, dn)


def _dsel_r(a, s, dn):
    ah, al = _split(a)
    return _d(ah, s, dn) + _d(al, s, dn)


_PAIRS = [(j, k) for j in range(8) for k in range(j, 8)]


def _sel_constants():
    sj = np.zeros((_B * _NPAIR_U, _B * _N), np.float32)
    sk = np.zeros((_B * _NPAIR_U, _B * _N), np.float32)
    em = np.zeros((_NPAIR_U, _N * _N), np.float32)
    for u, (j, k) in enumerate(_PAIRS):
        em[u, j * 8 + k] = 1.0
        em[u, k * 8 + j] = 1.0
        for i in range(_B):
            sj[i * _NPAIR_U + u, i * 8 + j] = 1.0
            sk[i * _NPAIR_U + u, i * 8 + k] = 1.0
    gs = np.zeros((128, 32), np.float32)
    for c in range(128):
        gs[c, c // 4] = 1.0
    return sj, sk, em, gs


def _clip1(x):
    return x - jax.nn.relu(x - 1.0)


def _build_pairs(base, sj, sk):
    dn = (((1,), (0,)), ((), ()))
    bh, bl = _split(base)
    pj = _d(sj, bh, dn) + _d(sj, bl, dn)
    pk = _d(sk, bh, dn) + _d(sk, bl, dn)
    return _clip1(pj + pk)


def _prep_body(m_ref, sj_ref, sk_ref, o_ref):
    m2 = m_ref[...]
    o_ref[0:16, :] = m2
    o_ref[16:_NV, :] = _build_pairs(m2, sj_ref[...], sk_ref[...])


def _gn_stats_from_sums(s1, s2, npos, gs):
    cnt = 4.0 * npos
    mu = _dsel_r(s1, gs, (((1,), (0,)), ((), ()))) / cnt
    ex2 = _dsel_r(s2, gs, (((1,), (0,)), ((), ()))) / cnt
    var = ex2 - mu * mu
    inv = jax.lax.rsqrt(var + 1e-5)
    mu_b = _dsel_r(mu, gs, (((1,), (1,)), ((), ())))
    inv_b = _dsel_r(inv, gs, (((1,), (1,)), ((), ())))
    return mu_b, inv_b


def _gn_apply_silu(y, mu_b, inv_b, g, bt):
    yn = (y - mu_b[:, None, :]) * inv_b[:, None, :]
    yn = yn * g[None, :, :] + bt[None, :, :]
    return yn * jax.nn.sigmoid(yn)


_CLASSES = [(cx, cy, cz) for cx in range(2) for cy in range(2)
            for cz in range(2)]
_CLS_N = [(4 - c[0]) * (4 - c[1]) * (4 - c[2]) for c in _CLASSES]
_CLS_OFF = [sum(_CLS_N[:i]) for i in range(len(_CLASSES))]

_POS_LIST = [(2 * ux + cx, 2 * uy + cy, 2 * uz + cz)
             for (cx, cy, cz) in _CLASSES
             for ux in range(4 - cx) for uy in range(4 - cy)
             for uz in range(4 - cz)]
_IM2COL_IDX = np.array(
    [[(2 * i + dx) * 256 + (2 * j + dy) * 16 + (2 * k + dz)
      for dx in range(3) for dy in range(3) for dz in range(3)]
     for (i, j, k) in _POS_LIST], dtype=np.int32)

_DN2 = (((2,), (0,)), ((), ()))


def _mapper_body(x1h_ref, x1l_ref, w1h_ref, w1l_ref, w2h_ref, w2l_ref,
                 w3h_ref, w3l_ref, prm_ref, gs_ref, o_ref, sc1, sc2):
    vb = x1h_ref.shape[0]
    prm = prm_ref[...]
    gs = gs_ref[...]

    y1 = _dmm(x1h_ref[...], x1l_ref[...], w1h_ref[...], w1l_ref[...], _DN2)
    sc1[...] = y1 + prm[0:1, :][None]

    def stats(ref, npos):
        s1 = jnp.zeros((vb, 128), _F32)
        s2 = jnp.zeros((vb, 128), _F32)
        for ci in range(8):
            off, n = _CLS_OFF[ci], _CLS_N[ci]
            y = ref[:, off:off + n, :]
            s1 = s1 + y.sum(axis=1)
            s2 = s2 + (y * y).sum(axis=1)
        return _gn_stats_from_sums(s1, s2, npos, gs)

    mu_b, inv_b = stats(sc1, 343.0)
    w2h, w2l = w2h_ref[...], w2l_ref[...]
    for ci in range(8):
        off, n = _CLS_OFF[ci], _CLS_N[ci]
        y = _gn_apply_silu(sc1[:, off:off + n, :], mu_b, inv_b,
                           prm[1:2, :], prm[2:3, :])
        yh, yl = _split(y)
        y2c = _dmm(yh, yl, w2h, w2l, _DN2)
        sc2[:, off:off + n, :] = y2c + prm[3:4, :][None]

    mu_b, inv_b = stats(sc2, 343.0)
    acc = jnp.zeros((vb, 27, 128), _F32)
    for ci, (cx, cy, cz) in enumerate(_CLASSES):
        off, n = _CLS_OFF[ci], _CLS_N[ci]
        lx, ly, lz = 4 - cx, 4 - cy, 4 - cz
        y = _gn_apply_silu(sc2[:, off:off + n, :], mu_b, inv_b,
                           prm[4:5, :], prm[5:6, :])
        yh, yl = _split(y)
        yh = yh.reshape(vb, lx, ly, lz, 128)
        yl = yl.reshape(vb, lx, ly, lz, 128)
        for dx in range(cx, 3, 2):
            for dy in range(cy, 3, 2):
                for dz in range(cz, 3, 2):
                    t = (dx * 3 + dy) * 3 + dz
                    ix = (slice(None), slice(dx // 2, dx // 2 + 3),
                          slice(dy // 2, dy // 2 + 3),
                          slice(dz // 2, dz // 2 + 3), slice(None))
                    acc = acc + _dmm(
                        yh[ix].reshape(vb, 27, 128),
                        yl[ix].reshape(vb, 27, 128),
                        w3h_ref[t], w3l_ref[t], _DN2)
    y3 = acc + prm[6:7, :][None]
    s1 = y3.sum(axis=1)
    s2 = (y3 * y3).sum(axis=1)
    mu_b, inv_b = _gn_stats_from_sums(s1, s2, 27.0, gs)
    y3 = _gn_apply_silu(y3, mu_b, inv_b, prm[7:8, :], prm[8:9, :])
    o_ref[...] = jnp.mean(y3, axis=1)


def _codebook_p(q, cb):
    d = q[:, None, :] - cb[None, :, :]
    sq = jnp.sum(d * d, axis=-1)
    return jnp.exp(-jnp.sqrt(jnp.maximum(sq, 1e-12)))


def _head_body(with_vox, f_ref, base_ref, qswh_ref, qswl_ref, qsb_ref,
               qpwh_ref, qpwl_ref, qpb_ref,
               pa_ref, pb_ref, sj_ref, sk_ref, em_ref, *out_refs):
    f = f_ref[...]
    base = base_ref[...]
    dn_t = (((1,), (1,)), ((), ()))
    f1h, f1l = _split(f[0:16, :])
    f2h, f2l = _split(f[16:_NV, :])
    q1 = _dmm(f1h, f1l, qswh_ref[...], qswl_ref[...], dn_t) + qsb_ref[...]
    q2 = _dmm(f2h, f2l, qpwh_ref[...], qpwl_ref[...], dn_t) + qpb_ref[...]
    pa = pa_ref[...]
    p1 = _codebook_p(q1, pa)
    out_refs[0][...] = _dot3(p1, pa, (((1,), (0,)), ((), ())))
    d2 = q2[:, None, :] - pb_ref[...][None, :, :]
    sq2 = jnp.sum(d2 * d2, axis=-1)
    p2c = jnp.exp(-jnp.sqrt(jnp.maximum(sq2, 1e-12)))
    ci = jax.lax.broadcasted_iota(jnp.int32, (12, 6), 0)
    gi = jax.lax.broadcasted_iota(jnp.int32, (12, 6), 1)
    sum2c = ((ci // 2) == gi).astype(_BF16)
    s2c = _dsel_r(p2c, sum2c, (((1,), (0,)), ((), ())))
    cols = [s2c[:, i:i + 1] for i in range(6)]
    num_c = cols[0]
    den_c = cols[0]
    for r in cols[1:]:
        num_c = jnp.maximum(num_c, r)
        den_c = den_c + r
    em = em_ref[...]
    nm = _dsel_r(num_c.reshape(_B, _NPAIR_U), em,
                 (((1,), (0,)), ((), ())))
    dn = _dsel_r(den_c.reshape(_B, _NPAIR_U), em,
                 (((1,), (0,)), ((), ())))
    pm = (nm / dn).reshape(_B, 8, 8)
    z8 = jnp.zeros((8, 8), _F32)
    p2m16 = jnp.concatenate(
        [jnp.concatenate([pm[0], z8], axis=1),
         jnp.concatenate([z8, pm[1]], axis=1)], axis=0)
    g2 = _dot3(p2m16, base, (((1,), (0,)), ((), ())))
    out_refs[1][...] = g2
    if with_vox:
        out_refs[2][0:16, :] = g2
        out_refs[2][16:_NV, :] = _build_pairs(g2, sj_ref[...], sk_ref[...])


def _run_mapper(vox, w1s, w2s, w3s, prm, gs):
    x1 = vox[:, jnp.asarray(_IM2COL_IDX)]
    x1h, x1l = _split(x1)
    nblk = _NV // _VB
    bs_w = [
        pl.BlockSpec((27, 128), lambda i: (0, 0)),
        pl.BlockSpec((27, 128), lambda i: (0, 0)),
        pl.BlockSpec((128, 128), lambda i: (0, 0)),
        pl.BlockSpec((128, 128), lambda i: (0, 0)),
        pl.BlockSpec((27, 128, 128), lambda i: (0, 0, 0)),
        pl.BlockSpec((27, 128, 128), lambda i: (0, 0, 0)),
        pl.BlockSpec((9, 128), lambda i: (0, 0)),
        pl.BlockSpec((128, 32), lambda i: (0, 0)),
    ]
    return pl.pallas_call(
        _mapper_body,
        grid=(nblk,),
        in_specs=[
            pl.BlockSpec((_VB, 343, 27), lambda i: (i, 0, 0)),
            pl.BlockSpec((_VB, 343, 27), lambda i: (i, 0, 0)),
        ] + bs_w,
        out_specs=pl.BlockSpec((_VB, 128), lambda i: (i, 0)),
        out_shape=jax.ShapeDtypeStruct((_NV, 128), _F32),
        scratch_shapes=[pltpu.VMEM((_VB, 343, 128), _F32),
                        pltpu.VMEM((_VB, 343, 128), _F32)],
    )(x1h, x1l, *w1s, *w2s, *w3s, prm, gs)


def _run_head(with_vox, f, base, qsw_s, qsb, qpw_s, qpb, pa, pb, sj, sk, em):
    outs = [jax.ShapeDtypeStruct((16, 256), _F32),
            jax.ShapeDtypeStruct((16, _VOX), _F32)]
    if with_vox:
        outs.append(jax.ShapeDtypeStruct((_NV, _VOX), _F32))
    return pl.pallas_call(
        functools.partial(_head_body, with_vox),
        out_shape=outs,
    )(f, base, *qsw_s, qsb, *qpw_s, qpb, pa, pb, sj, sk, em)


def kernel(m, W1, b1, g1, bt1, W2, b2, g2, bt2, W3, b3, g3, bt3,
           Q1_W, Q1_b, Q2_W, Q2_b, QH1_W, QH1_b, QH2_W, QH2_b,
           P1, P2, PH1, PH2):
    m2d = m.reshape(_B * _N, _VOX)
    w1s = _split(W1.transpose(2, 3, 4, 1, 0).reshape(27, 128))
    w2s = _split(W2[:, :, 0, 0, 0].transpose(1, 0))
    w3s = _split(W3.transpose(2, 3, 4, 1, 0).reshape(27, 128, 128))
    prm = jnp.stack([b1, g1, bt1, b2, g2, bt2, b3, g3, bt3])
    sj_np, sk_np, em_np, gs_np = _sel_constants()
    sj = jnp.asarray(sj_np, _BF16)
    sk = jnp.asarray(sk_np, _BF16)
    em = jnp.asarray(em_np, _BF16)
    gs = jnp.asarray(gs_np, _BF16)
    q1s, q2s = _split(Q1_W), _split(Q2_W)
    qh1s, qh2s = _split(QH1_W), _split(QH2_W)
    q1b, q2b = Q1_b.reshape(1, 256), Q2_b.reshape(1, 256)
    qh1b, qh2b = QH1_b.reshape(1, 256), QH2_b.reshape(1, 256)

    vox_a = pl.pallas_call(
        _prep_body,
        out_shape=jax.ShapeDtypeStruct((_NV, _VOX), _F32),
    )(m2d, sj, sk)
    feats_a = _run_mapper(vox_a, w1s, w2s, w3s, prm, gs)
    pr16, g2d, vox_b = _run_head(True, feats_a, m2d, q1s, q1b, q2s, q2b,
                                 P1, P2, sj, sk, em)
    feats_b = _run_mapper(vox_b, w1s, w2s, w3s, prm, gs)
    phr16, gh2d = _run_head(False, feats_b, g2d, qh1s, qh1b, qh2s, qh2b,
                            PH1, PH2, sj, sk, em)

    pr = pr16.reshape(_B, _N, 256)
    gr = g2d.reshape(_B, _N, 1, 16, 16, 16)
    phr = phr16.reshape(_B, _N, 256)
    ghr = gh2d.reshape(_B, _N, 1, 16, 16, 16)
    return pr, gr, phr, ghr
